# baseline (speedup 1.0000x reference)
import functools

import jax
import jax.numpy as jnp
from jax import lax
from jax.experimental import pallas as pl
from jax.experimental.pallas import tpu as pltpu
from jax.experimental.pallas import tpu_sc as plsc

_NC = 2
_NS = 16
_CG = 64


def _sc_gather_body(b_per_w, idx4_hbm, idxp_hbm, mu_hbm, ls_hbm, phi_hbm,
                    mu_out, ls_out, phi_out,
                    idx4_v, idxp_v, buf0, buf1, buf2, buf3, sem, sem2):
    bufs = (buf0, buf1, buf2, buf3)
    wid = lax.axis_index("s") * _NC + lax.axis_index("c")
    base = wid * b_per_w
    pltpu.sync_copy(idx4_hbm.at[pl.ds(base, b_per_w)], idx4_v)
    pltpu.sync_copy(idxp_hbm.at[pl.ds(base, b_per_w)], idxp_v)

    n_per_tab = b_per_w // _CG
    plan = [(mu_hbm, mu_out, idx4_v), (ls_hbm, ls_out, idx4_v),
            (phi_hbm, phi_out, idxp_v)]
    n_tot = 3 * n_per_tab
    nb = len(bufs)
    lookahead = 2
    gather_h = {}
    write_h = {}
    for i in range(n_tot + lookahead):
        if i < n_tot:
            tab, _, idx_v = plan[i // n_per_tab]
            lo = (i % n_per_tab) * _CG
            if i >= nb:
                write_h[i - nb].wait()
            gather_h[i] = pltpu.async_copy(
                tab.at[idx_v.at[pl.ds(lo, _CG)]], bufs[i % nb], sem)
        j = i - lookahead
        if 0 <= j < n_tot:
            _, out, _ = plan[j // n_per_tab]
            lo = (j % n_per_tab) * _CG
            gather_h[j].wait()
            write_h[j] = pltpu.async_copy(
                bufs[j % nb], out.at[pl.ds(base + lo, _CG)], sem2)
    for j in range(n_tot - nb, n_tot):
        write_h[j].wait()


def _sc_gather(idx4, idxp, mu_w, ls_w, phi_w):
    n = idx4.shape[0]
    nw = _NC * _NS
    b_per_w = n // nw
    mesh = plsc.VectorSubcoreMesh(core_axis_name="c", subcore_axis_name="s",
                                  num_cores=_NC, num_subcores=_NS)
    kern = pl.kernel(
        functools.partial(_sc_gather_body, b_per_w),
        out_type=(
            jax.ShapeDtypeStruct((n, 128), jnp.float32),
            jax.ShapeDtypeStruct((n, 128), jnp.float32),
            jax.ShapeDtypeStruct((n, 128), jnp.float32),
        ),
        mesh=mesh,
        scratch_types=[
            pltpu.VMEM((b_per_w,), jnp.int32),
            pltpu.VMEM((b_per_w,), jnp.int32),
            pltpu.VMEM((_CG, 128), jnp.float32),
            pltpu.VMEM((_CG, 128), jnp.float32),
            pltpu.VMEM((_CG, 128), jnp.float32),
            pltpu.VMEM((_CG, 128), jnp.float32),
            pltpu.SemaphoreType.DMA,
            pltpu.SemaphoreType.DMA,
        ],
        compiler_params=pltpu.CompilerParams(use_tc_tiling_on_sc=True,
                                             needs_layout_passes=False),
    )
    return kern(idx4, idxp, mu_w, ls_w, phi_w)


def _extract_body(mu_ref, ls_ref, phi_ref, sub4_ref, subp_ref,
                  mu_out, sig_out, phi_out):
    t, w = mu_ref.shape
    k = 32
    c = lax.broadcasted_iota(jnp.int32, (t, w), 1)
    zero = jnp.float32(0.0)
    one = jnp.float32(1.0)

    mask4 = (c >> 5) == sub4_ref[...]
    cr = lax.broadcasted_iota(jnp.int32, (w, k), 0)
    kr = lax.broadcasted_iota(jnp.int32, (w, k), 1)
    r32 = jnp.where((cr & (k - 1)) == kr, one, zero)
    mu_out[...] = jnp.dot(jnp.where(mask4, mu_ref[...], zero), r32,
                          preferred_element_type=jnp.float32,
                          precision=lax.Precision.HIGHEST)
    ls_sel = jnp.dot(jnp.where(mask4, ls_ref[...], zero), r32,
                     preferred_element_type=jnp.float32,
                          precision=lax.Precision.HIGHEST)
    sig = jnp.exp(ls_sel)
    je = lax.broadcasted_iota(jnp.int32, (k, k * k), 0)
    ce = lax.broadcasted_iota(jnp.int32, (k, k * k), 1)
    e = jnp.where(ce == (k + 1) * je, one, zero)
    sig_out[...] = jnp.dot(sig, e, preferred_element_type=jnp.float32,
                          precision=lax.Precision.HIGHEST)

    maskp = (c >> 2) == subp_ref[...]
    cp = lax.broadcasted_iota(jnp.int32, (w, 4), 0)
    kp = lax.broadcasted_iota(jnp.int32, (w, 4), 1)
    r4 = jnp.where((cp & 3) == kp, one, zero)
    phi_out[...] = jnp.dot(jnp.where(maskp, phi_ref[...], zero), r4,
                           preferred_element_type=jnp.float32,
                          precision=lax.Precision.HIGHEST)


def _extract_expand(mu128, ls128, phi128, sub4, subp, block):
    n, w = mu128.shape
    k = 32
    grid = n // block
    row_spec = pl.BlockSpec((block, w), lambda i: (i, 0))
    sub_spec = pl.BlockSpec((block, 1), lambda i: (i, 0))
    return pl.pallas_call(
        _extract_body,
        grid=(grid,),
        in_specs=[row_spec, row_spec, row_spec, sub_spec, sub_spec],
        out_specs=[
            pl.BlockSpec((block, k), lambda i: (i, 0)),
            pl.BlockSpec((block, k * k), lambda i: (i, 0)),
            pl.BlockSpec((block, 4), lambda i: (i, 0)),
        ],
        out_shape=[
            jax.ShapeDtypeStruct((n, k), jnp.float32),
            jax.ShapeDtypeStruct((n, k * k), jnp.float32),
            jax.ShapeDtypeStruct((n, 4), jnp.float32),
        ],
    )(mu128, ls128, phi128, sub4, subp)


def kernel(token_ids, mu_table, log_sigma_diag, phi_table):
    b, l = token_ids.shape
    v, k = mu_table.shape
    p = phi_table.shape[1]
    n = b * l
    tok = token_ids.reshape(n).astype(jnp.int32)
    idx4 = tok // 4
    idxp = tok // 32
    sub4 = (tok % 4).reshape(n, 1)
    subp = (tok % 32).reshape(n, 1)
    mu_w = mu_table.reshape(v * k // 128, 128)
    ls_w = log_sigma_diag.reshape(v * k // 128, 128)
    phi_pad = jnp.pad(phi_table, ((0, 0), (0, 1)))
    phi_w = phi_pad.reshape(v * 4 // 128, 128)
    mu128, ls128, phi128 = _sc_gather(idx4, idxp, mu_w, ls_w, phi_w)
    mu_flat, sigma_rows, phi4 = _extract_expand(
        mu128, ls128, phi128, sub4, subp, block=256)
    mu = mu_flat.reshape(b, l, k)
    sigma = sigma_rows.reshape(b, l, k, k)
    phi = phi4[:, :p].reshape(b, l, p)
    return (mu, sigma, phi)
